# Initial kernel scaffold; baseline (speedup 1.0000x reference)
import jax
import jax.numpy as jnp
from jax.experimental import pallas as pl


def kernel(x, edge_index, W, a):
    raise NotImplementedError("write your pallas kernel here")


# speedup vs baseline: 6.2051x; 6.2051x over previous
import functools

import jax
import jax.numpy as jnp
from jax import lax
from jax.experimental import pallas as pl
from jax.experimental.pallas import tpu as pltpu
from jax.experimental.pallas import tpu_sc as plsc

N = 10000
E = 320000
D = 128
NPAD = 10240
NC = 2
NS = 16
NW = NC * NS
L = 16
DR = NPAD // D

EPAD = 327680
EPT = EPAD // NW
K = 64
CB = EPT // K
RPT = NPAD // NS
DRPT = 8


def _tc_project(x, W, A2):
    BR = 200

    def body(x_ref, w_ref, a2_ref, z_ref, s_ref):
        z = jnp.dot(x_ref[...], w_ref[...], preferred_element_type=jnp.float32)
        z_ref[...] = z
        s_ref[...] = jnp.dot(z, a2_ref[...], preferred_element_type=jnp.float32)

    return pl.pallas_call(
        body,
        grid=(N // BR,),
        in_specs=[
            pl.BlockSpec((BR, D), lambda i: (i, 0)),
            pl.BlockSpec((D, D), lambda i: (0, 0)),
            pl.BlockSpec((D, 2), lambda i: (0, 0)),
        ],
        out_specs=[
            pl.BlockSpec((BR, D), lambda i: (i, 0)),
            pl.BlockSpec((BR, 2), lambda i: (i, 0)),
        ],
        out_shape=[
            jax.ShapeDtypeStruct((N, D), jnp.float32),
            jax.ShapeDtypeStruct((N, 2), jnp.float32),
        ],
    )(x, W, A2)


def _sc_attention(z, s12, src, dst):
    mesh = plsc.VectorSubcoreMesh(core_axis_name="c", subcore_axis_name="s")

    @functools.partial(
        pl.kernel,
        out_type=[
            jax.ShapeDtypeStruct((NC, NPAD, D), jnp.float32),
            jax.ShapeDtypeStruct((NC, DR, D), jnp.float32),
        ],
        mesh=mesh,
        compiler_params=pltpu.CompilerParams(needs_layout_passes=False),
        scratch_types=[
            pltpu.VMEM((2 * N + 32,), jnp.float32),
            pltpu.VMEM((K,), jnp.int32),
            pltpu.VMEM((K,), jnp.int32),
            pltpu.VMEM((K,), jnp.int32),
            pltpu.VMEM((K,), jnp.int32),
            pltpu.VMEM((K,), jnp.float32),
            pltpu.VMEM((K, D), jnp.float32),
            pltpu.VMEM((K, D), jnp.float32),
            pltpu.VMEM_SHARED((NPAD, D), jnp.float32),
            pltpu.VMEM_SHARED((DR, D), jnp.float32),
            pltpu.SemaphoreType.DMA,
        ],
    )
    def k(z_hbm, s_hbm, src_hbm, dst_hbm, h_out, d_out,
          s12_v, srcb, dstb, dhib, dlob, wbuf, rows, drows, h_sh, d_sh, sem):
        cid = lax.axis_index("c")
        sid = lax.axis_index("s")
        wid = sid * NC + cid

        pltpu.sync_copy(s_hbm, s12_v)

        zv = jnp.zeros((L,), jnp.float32)
        lane = lax.iota(jnp.int32, L)
        one_i = jnp.ones((L,), jnp.int32)

        def zrow(r, c):
            for v in range(D // L):
                rows[r, pl.ds(v * L, L)] = zv
                drows[r, pl.ds(v * L, L)] = zv
            return c
        lax.fori_loop(0, K, zrow, 0)
        own = sid * RPT
        for t in range(RPT // K):
            pltpu.sync_copy(rows, h_sh.at[pl.ds(own + t * K, K)])
        @pl.when(sid < DR // DRPT)
        def _():
            pltpu.sync_copy(drows.at[pl.ds(0, DRPT)],
                            d_sh.at[pl.ds(sid * DRPT, DRPT)])
        plsc.subcore_barrier()

        base = wid * EPT

        def chunk(j, c):
            off = base + j * K
            pltpu.sync_copy(src_hbm.at[pl.ds(off, K)], srcb)
            pltpu.sync_copy(dst_hbm.at[pl.ds(off, K)], dstb)
            cp = pltpu.async_copy(z_hbm.at[srcb], rows, sem)

            def grp(i, cc):
                sl = pl.ds(i * L, L)
                sv = srcb[sl]
                dv = dstb[sl]
                e = (plsc.load_gather(s12_v, [sv + sv])
                     + plsc.load_gather(s12_v, [dv + dv + one_i]))
                e = jnp.where(e >= 0.0, e, 0.01 * e)
                wbuf[sl] = jnp.exp(e)
                dhib[sl] = lax.shift_right_logical(dv, 7)
                dlob[sl] = lax.bitwise_and(dv, jnp.full((L,), 127, jnp.int32))
                return cc
            lax.fori_loop(0, K // L, grp, 0)

            cp.wait()

            def rowloop(r, c2):
                ridx = jnp.full((L,), 0, jnp.int32) + r
                wv = plsc.load_gather(wbuf, [ridx])
                dlov = plsc.load_gather(dlob, [ridx])
                for v in range(D // L):
                    rsl = pl.ds(v * L, L)
                    rows[r, rsl] = rows[r, rsl] * wv
                    drows[r, rsl] = jnp.where(lane + (v * L) == dlov, wv, 0.0)
                return c2
            lax.fori_loop(0, K, rowloop, 0)

            pltpu.sync_copy(rows, h_sh.at[dstb], add=True)
            pltpu.sync_copy(drows, d_sh.at[dhib], add=True)
            return c
        lax.fori_loop(0, CB, chunk, 0)

        plsc.subcore_barrier()

        osl = pl.ds(own, RPT)
        pltpu.sync_copy(h_sh.at[osl], h_out.at[cid, osl])

        @pl.when(sid < DR // DRPT)
        def _():
            dsl = pl.ds(sid * DRPT, DRPT)
            pltpu.sync_copy(d_sh.at[dsl], d_out.at[cid, dsl])

    return k(z, s12, src, dst)


def _tc_combine(hpart, dpart):
    BR = 256

    def body(hp_ref, dp_ref, out_ref):
        dsum = dp_ref[0] + dp_ref[1]
        out_ref[...] = (hp_ref[0] + hp_ref[1]) / (dsum + 1e-16)

    return pl.pallas_call(
        body,
        grid=(NPAD // BR,),
        in_specs=[
            pl.BlockSpec((NC, BR, D), lambda i: (0, i, 0)),
            pl.BlockSpec((NC, BR, 1), lambda i: (0, i, 0)),
        ],
        out_specs=pl.BlockSpec((BR, D), lambda i: (i, 0)),
        out_shape=jax.ShapeDtypeStruct((NPAD, D), jnp.float32),
    )(hpart, dpart)


def kernel(x, edge_index, W, a):
    src = edge_index[0].astype(jnp.int32)
    dst = edge_index[1].astype(jnp.int32)
    npad_e = EPAD - E
    src = jnp.concatenate([src, jnp.zeros((npad_e,), jnp.int32)])
    dst = jnp.concatenate([dst, jnp.full((npad_e,), N, jnp.int32)])
    A2 = a.reshape(2, D).T
    z, s12 = _tc_project(x, W, A2)
    s12_flat = jnp.pad(s12.reshape(-1), (0, 32))
    hpart, dpart = _sc_attention(z, s12_flat, src, dst)
    dflat = dpart.reshape(NC, NPAD, 1)
    return _tc_combine(hpart, dflat)[:N]


# speedup vs baseline: 8.1390x; 1.3117x over previous
import functools

import jax
import jax.numpy as jnp
from jax import lax
from jax.experimental import pallas as pl
from jax.experimental.pallas import tpu as pltpu
from jax.experimental.pallas import tpu_sc as plsc

N = 10000
E = 320000
D = 128
NPAD = 10240
NC = 2
NS = 16
NW = NC * NS
L = 16
DR = NPAD // D

EPAD = 327680
EPT = EPAD // NW
K = 32
CB = EPT // K
RPT = NPAD // NS
DRPT = 8


def _tc_project(x, W, A2):
    BR = 200

    def body(x_ref, w_ref, a2_ref, z_ref, s_ref):
        z = jnp.dot(x_ref[...], w_ref[...], preferred_element_type=jnp.float32)
        z_ref[...] = z
        s_ref[...] = jnp.dot(z, a2_ref[...], preferred_element_type=jnp.float32)

    return pl.pallas_call(
        body,
        grid=(N // BR,),
        in_specs=[
            pl.BlockSpec((BR, D), lambda i: (i, 0)),
            pl.BlockSpec((D, D), lambda i: (0, 0)),
            pl.BlockSpec((D, 2), lambda i: (0, 0)),
        ],
        out_specs=[
            pl.BlockSpec((BR, D), lambda i: (i, 0)),
            pl.BlockSpec((BR, 2), lambda i: (i, 0)),
        ],
        out_shape=[
            jax.ShapeDtypeStruct((N, D), jnp.float32),
            jax.ShapeDtypeStruct((N, 2), jnp.float32),
        ],
    )(x, W, A2)


def _sc_attention(z, s12, src, dst):
    mesh = plsc.VectorSubcoreMesh(core_axis_name="c", subcore_axis_name="s")

    @functools.partial(
        pl.kernel,
        out_type=[
            jax.ShapeDtypeStruct((NC, NPAD, D), jnp.float32),
            jax.ShapeDtypeStruct((NC, DR, D), jnp.float32),
        ],
        mesh=mesh,
        compiler_params=pltpu.CompilerParams(needs_layout_passes=False),
        scratch_types=[
            pltpu.VMEM((2 * N + 32,), jnp.float32),
            pltpu.VMEM((K,), jnp.int32),
            pltpu.VMEM((K,), jnp.int32),
            pltpu.VMEM((K,), jnp.int32),
            pltpu.VMEM((K,), jnp.int32),
            pltpu.VMEM((K,), jnp.int32),
            pltpu.VMEM((K,), jnp.float32),
            pltpu.VMEM((K, D), jnp.float32),
            pltpu.VMEM((K, D), jnp.float32),
            pltpu.VMEM((K, D), jnp.float32),
            pltpu.VMEM_SHARED((NPAD, D), jnp.float32),
            pltpu.VMEM_SHARED((DR, D), jnp.float32),
            pltpu.SemaphoreType.DMA,
            pltpu.SemaphoreType.DMA,
        ],
    )
    def k(z_hbm, s_hbm, src_hbm, dst_hbm, h_out, d_out,
          s12_v, srcb0, srcb1, dstb0, dstb1, dhib, wbuf,
          rows0, rows1, drows, h_sh, d_sh, sem0, sem1):
        cid = lax.axis_index("c")
        sid = lax.axis_index("s")
        wid = sid * NC + cid

        pltpu.sync_copy(s_hbm, s12_v)

        zv = jnp.zeros((L,), jnp.float32)
        lane = lax.iota(jnp.int32, L)
        one_i = jnp.ones((L,), jnp.int32)

        def zrow(r, c):
            for v in range(D // L):
                rows0[r, pl.ds(v * L, L)] = zv
                drows[r, pl.ds(v * L, L)] = zv
            return c
        lax.fori_loop(0, K, zrow, 0)
        own = sid * RPT
        for t in range(RPT // K):
            pltpu.sync_copy(rows0, h_sh.at[pl.ds(own + t * K, K)])
        @pl.when(sid < DR // DRPT)
        def _():
            pltpu.sync_copy(drows.at[pl.ds(0, DRPT)],
                            d_sh.at[pl.ds(sid * DRPT, DRPT)])
        plsc.subcore_barrier()

        base = wid * EPT
        srcb = (srcb0, srcb1)
        dstb = (dstb0, dstb1)
        rows = (rows0, rows1)
        sem = (sem0, sem1)
        c127 = jnp.full((L,), 127, jnp.int32)

        def load_issue(j, b):
            off = j * K + base
            pltpu.sync_copy(src_hbm.at[pl.ds(off, K)], srcb[b])
            pltpu.sync_copy(dst_hbm.at[pl.ds(off, K)], dstb[b])
            pltpu.async_copy(z_hbm.at[srcb[b]], rows[b], sem[b])

        def process(b):
            def grp(i, cc):
                sl = pl.ds(i * L, L)
                sv = srcb[b][sl]
                dv = dstb[b][sl]
                e = (plsc.load_gather(s12_v, [sv + sv])
                     + plsc.load_gather(s12_v, [dv + dv + one_i]))
                e = jnp.where(e >= 0.0, e, 0.01 * e)
                wbuf[sl] = jnp.exp(e)
                dhib[sl] = lax.shift_right_logical(dv, 7)
                return cc
            lax.fori_loop(0, K // L, grp, 0)

            pltpu.make_async_copy(z_hbm.at[srcb[b]], rows[b], sem[b]).wait()

            def rowloop(r, c2):
                ridx = jnp.full((L,), 0, jnp.int32) + r
                wv = plsc.load_gather(wbuf, [ridx])
                dlov = lax.bitwise_and(plsc.load_gather(dstb[b], [ridx]), c127)
                for v in range(D // L):
                    rsl = pl.ds(v * L, L)
                    rows[b][r, rsl] = rows[b][r, rsl] * wv
                    drows[r, rsl] = jnp.where(lane + (v * L) == dlov, wv, 0.0)
                return c2
            lax.fori_loop(0, K, rowloop, 0)

            pltpu.sync_copy(rows[b], h_sh.at[dstb[b]], add=True)
            pltpu.sync_copy(drows, d_sh.at[dhib], add=True)

        load_issue(0, 0)
        HB = CB // 2

        def pair(i, c):
            load_issue(2 * i + 1, 1)
            process(0)

            @pl.when(i < HB - 1)
            def _():
                load_issue(2 * i + 2, 0)
            process(1)
            return c
        lax.fori_loop(0, HB, pair, 0)

        plsc.subcore_barrier()

        osl = pl.ds(own, RPT)
        pltpu.sync_copy(h_sh.at[osl], h_out.at[cid, osl])

        @pl.when(sid < DR // DRPT)
        def _():
            dsl = pl.ds(sid * DRPT, DRPT)
            pltpu.sync_copy(d_sh.at[dsl], d_out.at[cid, dsl])

    return k(z, s12, src, dst)


def _tc_combine(hpart, dpart):
    BR = 256

    def body(hp_ref, dp_ref, out_ref):
        dsum = dp_ref[0] + dp_ref[1]
        out_ref[...] = (hp_ref[0] + hp_ref[1]) / (dsum + 1e-16)

    return pl.pallas_call(
        body,
        grid=(NPAD // BR,),
        in_specs=[
            pl.BlockSpec((NC, BR, D), lambda i: (0, i, 0)),
            pl.BlockSpec((NC, BR, 1), lambda i: (0, i, 0)),
        ],
        out_specs=pl.BlockSpec((BR, D), lambda i: (i, 0)),
        out_shape=jax.ShapeDtypeStruct((NPAD, D), jnp.float32),
    )(hpart, dpart)


def kernel(x, edge_index, W, a):
    src = edge_index[0].astype(jnp.int32)
    dst = edge_index[1].astype(jnp.int32)
    npad_e = EPAD - E
    src = jnp.concatenate([src, jnp.zeros((npad_e,), jnp.int32)])
    dst = jnp.concatenate([dst, jnp.full((npad_e,), N, jnp.int32)])
    A2 = a.reshape(2, D).T
    z, s12 = _tc_project(x, W, A2)
    s12_flat = jnp.pad(s12.reshape(-1), (0, 32))
    hpart, dpart = _sc_attention(z, s12_flat, src, dst)
    dflat = dpart.reshape(NC, NPAD, 1)
    return _tc_combine(hpart, dflat)[:N]
